# speedup vs baseline: 1.2846x; 1.2846x over previous
import jax
import jax.numpy as jnp
from jax.experimental import pallas as pl

_G, _S, _E, _CAP = 4, 2048, 8, 512
_K = _E * _CAP
_SB = 256


def _body(o_ref, b_ref):
    j = pl.program_id(1)
    s = jax.lax.broadcasted_iota(jnp.int32, (1, _SB, _K), 1) + j * _SB
    k = jax.lax.broadcasted_iota(jnp.int32, (1, _SB, _K), 2)
    hit = k == (s % _E) * _CAP + s // _E
    o_ref[...] = hit.astype(jnp.float32)
    b_ref[...] = hit


def kernel(input):
    out3, bool3 = pl.pallas_call(
        _body,
        grid=(_G, _S // _SB),
        out_specs=[
            pl.BlockSpec((1, _SB, _K), lambda i, j: (i, j, 0)),
            pl.BlockSpec((1, _SB, _K), lambda i, j: (i, j, 0)),
        ],
        out_shape=[
            jax.ShapeDtypeStruct((_G, _S, _K), jnp.float32),
            jax.ShapeDtypeStruct((_G, _S, _K), jnp.bool_),
        ],
    )()
    out = out3.reshape(_G, _S, _E, _CAP)
    return (0.0, out, bool3.reshape(_G, _S, _E, _CAP))


# speedup vs baseline: 2.7003x; 2.1020x over previous
import jax
import jax.numpy as jnp
from jax.experimental import pallas as pl

_G, _S, _E, _CAP = 4, 2048, 8, 512
_SB = 256


def _body(o_ref, b_ref):
    j = pl.program_id(1)
    shp = (1, _SB, _E, _CAP)
    s = jax.lax.broadcasted_iota(jnp.int32, shp, 1) + j * _SB
    e = jax.lax.broadcasted_iota(jnp.int32, shp, 2)
    c = jax.lax.broadcasted_iota(jnp.int32, shp, 3)
    hit = (e == s % _E) & (c == s // _E)
    o_ref[...] = hit.astype(jnp.float32)
    b_ref[...] = hit


def kernel(input):
    out, boolout = pl.pallas_call(
        _body,
        grid=(_G, _S // _SB),
        out_specs=[
            pl.BlockSpec((1, _SB, _E, _CAP), lambda i, j: (i, j, 0, 0)),
            pl.BlockSpec((1, _SB, _E, _CAP), lambda i, j: (i, j, 0, 0)),
        ],
        out_shape=[
            jax.ShapeDtypeStruct((_G, _S, _E, _CAP), jnp.float32),
            jax.ShapeDtypeStruct((_G, _S, _E, _CAP), jnp.bool_),
        ],
    )()
    return (0.0, out, boolout)
